# baseline (speedup 1.0000x reference)
import functools

import jax
import jax.numpy as jnp
from jax import lax
from jax.experimental import pallas as pl
from jax.experimental.pallas import tpu as pltpu
from jax.experimental.pallas import tpu_sc as plsc

N_NODES = 10000
D = 128
E = 320000
N_LAYERS = 2

NW = 32
K = 64
CHUNKS_PER_W = (E + NW * K - 1) // (NW * K)
EP = NW * CHUNKS_PER_W * K
ROWS_PER_SUB = 632
ROWS_LAST = N_NODES - 15 * ROWS_PER_SUB
NGRP = K // 16


def _prep_body(x_ref, w1a_ref, w1b_ref, atab_ref, betab_ref):
    x = x_ref[...]
    atab_ref[...] = jnp.dot(x, w1a_ref[...], preferred_element_type=jnp.float32)
    b = jnp.dot(x, w1b_ref[...], preferred_element_type=jnp.float32)
    betab_ref[...] = jnp.concatenate([b, x], axis=1)


def _prep_tables(emb, w1a, w1b):
    bn = 2000
    return pl.pallas_call(
        _prep_body,
        grid=(N_NODES // bn,),
        in_specs=[pl.BlockSpec((bn, D), lambda i: (i, 0)),
                  pl.BlockSpec((D, D), lambda i: (0, 0)),
                  pl.BlockSpec((D, D), lambda i: (0, 0))],
        out_specs=[pl.BlockSpec((bn, D), lambda i: (i, 0)),
                   pl.BlockSpec((bn, 2 * D), lambda i: (i, 0))],
        out_shape=[jax.ShapeDtypeStruct((N_NODES, D), jnp.float32),
                   jax.ShapeDtypeStruct((N_NODES, 2 * D), jnp.float32)],
    )(emb, w1a, w1b)


def _mid_body(p_ref, w1a_ref, w1b_ref, emb_ref, atab_ref, betab_ref):
    x = p_ref[0] + p_ref[1]
    emb_ref[...] = x
    atab_ref[...] = jnp.dot(x, w1a_ref[...], preferred_element_type=jnp.float32)
    b = jnp.dot(x, w1b_ref[...], preferred_element_type=jnp.float32)
    betab_ref[...] = jnp.concatenate([b, x], axis=1)


def _mid_tables(partials, w1a, w1b):
    bn = 2000
    return pl.pallas_call(
        _mid_body,
        grid=(N_NODES // bn,),
        in_specs=[pl.BlockSpec((2, bn, D), lambda i: (0, i, 0)),
                  pl.BlockSpec((D, D), lambda i: (0, 0)),
                  pl.BlockSpec((D, D), lambda i: (0, 0))],
        out_specs=[pl.BlockSpec((bn, D), lambda i: (i, 0)),
                   pl.BlockSpec((bn, D), lambda i: (i, 0)),
                   pl.BlockSpec((bn, 2 * D), lambda i: (i, 0))],
        out_shape=[jax.ShapeDtypeStruct((N_NODES, D), jnp.float32),
                   jax.ShapeDtypeStruct((N_NODES, D), jnp.float32),
                   jax.ShapeDtypeStruct((N_NODES, 2 * D), jnp.float32)],
    )(partials, w1a, w1b)


def _final_body(e0_ref, e1_ref, p_ref, out_ref):
    out_ref[...] = (e0_ref[...] + e1_ref[...] + p_ref[0] + p_ref[1]) * (
        1.0 / (N_LAYERS + 1))


def _final_mean(emb0, emb1, partials2):
    bn = 2000
    return pl.pallas_call(
        _final_body,
        grid=(N_NODES // bn,),
        in_specs=[pl.BlockSpec((bn, D), lambda i: (i, 0)),
                  pl.BlockSpec((bn, D), lambda i: (i, 0)),
                  pl.BlockSpec((2, bn, D), lambda i: (0, i, 0))],
        out_specs=pl.BlockSpec((bn, D), lambda i: (i, 0)),
        out_shape=jax.ShapeDtypeStruct((N_NODES, D), jnp.float32),
    )(emb0, emb1, partials2)


def _make_edge_kernel(scale):
    mesh = plsc.VectorSubcoreMesh(core_axis_name="c", subcore_axis_name="s")

    @functools.partial(
        pl.kernel, mesh=mesh,
        compiler_params=pltpu.CompilerParams(needs_layout_passes=False),
        out_type=(jax.ShapeDtypeStruct((2, N_NODES, D), jnp.float32),
                  jax.ShapeDtypeStruct((EP,), jnp.float32)),
        scratch_types=[
            pltpu.VMEM_SHARED((N_NODES, D), jnp.float32),
            pltpu.VMEM((K,), jnp.int32),
            pltpu.VMEM((K,), jnp.int32),
            pltpu.VMEM((K,), jnp.float32),
            pltpu.VMEM((K,), jnp.float32),
            pltpu.VMEM((K, D), jnp.float32),
            pltpu.VMEM((K, 2 * D), jnp.float32),
            pltpu.VMEM((K, D), jnp.float32),
            pltpu.VMEM((D,), jnp.float32),
            pltpu.SemaphoreType.DMA,
            pltpu.SemaphoreType.DMA,
        ],
    )
    def edge_kernel(atab, betab, rows_hbm, cols_hbm, gv_hbm, w2_hbm, zeros_hbm,
                    outp, gv_out, acc, rows_v, cols_v, gv_v, gvn_v, S, T, M,
                    w2_v, sem_s, sem_t):
        c = lax.axis_index("c")
        s = lax.axis_index("s")
        w = c * 16 + s
        pltpu.sync_copy(w2_hbm, w2_v)

        @pl.when(s < 15)
        def _():
            pltpu.sync_copy(zeros_hbm,
                            acc.at[pl.ds(s * ROWS_PER_SUB, ROWS_PER_SUB)])

        @pl.when(s == 15)
        def _():
            pltpu.sync_copy(zeros_hbm.at[pl.ds(0, ROWS_LAST)],
                            acc.at[pl.ds(15 * ROWS_PER_SUB, ROWS_LAST)])

        plsc.subcore_barrier()

        erows = [lax.iota(jnp.int32, 16) + (g * 16) for g in range(NGRP)]

        def chunk_body(i, _carry):
            base = (w * CHUNKS_PER_W + i) * K
            pltpu.sync_copy(rows_hbm.at[pl.ds(base, K)], rows_v)
            pltpu.sync_copy(cols_hbm.at[pl.ds(base, K)], cols_v)
            pltpu.sync_copy(gv_hbm.at[pl.ds(base, K)], gv_v)
            cp_s = pltpu.async_copy(atab.at[rows_v], S, sem_s)
            cp_t = pltpu.async_copy(betab.at[cols_v], T, sem_t)
            cp_s.wait()
            cp_t.wait()

            def zbody(dd, accz):
                dsplat = jnp.broadcast_to(dd, (16,))
                w2d = plsc.load_gather(w2_v, [dsplat])
                out = []
                for g in range(NGRP):
                    a = plsc.load_gather(S, [erows[g], dsplat])
                    b = plsc.load_gather(T, [erows[g], dsplat])
                    out.append(accz[g] + jnp.maximum(a + b, 0.0) * w2d)
                return tuple(out)

            z0 = tuple(jnp.zeros((16,), jnp.float32) for _ in range(NGRP))
            zs = lax.fori_loop(0, D, zbody, z0)

            gvns = []
            for g in range(NGRP):
                gv_g = gv_v[pl.ds(g * 16, 16)]
                sig = 1.0 / (1.0 + jnp.exp(-zs[g]))
                gvn = gv_g * jnp.exp(sig * (-scale))
                gvn_v[pl.ds(g * 16, 16)] = gvn
                gvns.append(gvn)

            def mbody(dd, _c2):
                dsplat = jnp.broadcast_to(dd, (16,))
                tsplat = dsplat + D
                for g in range(NGRP):
                    t = plsc.load_gather(T, [erows[g], tsplat])
                    plsc.store_scatter(M, [erows[g], dsplat], t * gvns[g])
                return 0

            lax.fori_loop(0, D, mbody, 0)

            pltpu.sync_copy(gvn_v, gv_out.at[pl.ds(base, K)])
            pltpu.sync_copy(M, acc.at[rows_v], add=True)
            return 0

        lax.fori_loop(0, CHUNKS_PER_W, chunk_body, 0)
        plsc.subcore_barrier()

        @pl.when(s < 15)
        def _():
            pltpu.sync_copy(acc.at[pl.ds(s * ROWS_PER_SUB, ROWS_PER_SUB)],
                            outp.at[c, pl.ds(s * ROWS_PER_SUB, ROWS_PER_SUB)])

        @pl.when(s == 15)
        def _():
            pltpu.sync_copy(acc.at[pl.ds(15 * ROWS_PER_SUB, ROWS_LAST)],
                            outp.at[c, pl.ds(15 * ROWS_PER_SUB, ROWS_LAST)])

    return edge_kernel


_edge_kernels = [_make_edge_kernel(float(layer + 1)) for layer in range(N_LAYERS)]


def kernel(user_emb, item_emb, g_values, W1, W2, g_row, g_col):
    emb0 = jnp.concatenate([user_emb, item_emb], axis=0)
    w1a, w1b = W1[:D], W1[D:]
    w2 = jnp.reshape(W2, (D,))
    pad = EP - E
    rows = jnp.concatenate([g_row, jnp.zeros((pad,), jnp.int32)])
    cols = jnp.concatenate([g_col, jnp.zeros((pad,), jnp.int32)])
    gv = jnp.concatenate([g_values, jnp.zeros((pad,), jnp.float32)])
    zeros = jnp.zeros((ROWS_PER_SUB, D), jnp.float32)

    atab, betab = _prep_tables(emb0, w1a, w1b)
    partials1, gv = _edge_kernels[0](atab, betab, rows, cols, gv, w2, zeros)
    emb1, atab, betab = _mid_tables(partials1, w1a, w1b)
    partials2, gv = _edge_kernels[1](atab, betab, rows, cols, gv, w2, zeros)
    return _final_mean(emb0, emb1, partials2)


# speedup vs baseline: 1.1996x; 1.1996x over previous
import functools

import jax
import jax.numpy as jnp
from jax import lax
from jax.experimental import pallas as pl
from jax.experimental.pallas import tpu as pltpu
from jax.experimental.pallas import tpu_sc as plsc

N_NODES = 10000
D = 128
E = 320000
N_LAYERS = 2

NW = 32
K = 48
NCH = 212
EP = NW * NCH * K
NGRP = K // 16
ROWS_PER_SUB = 632
ROWS_LAST = N_NODES - 15 * ROWS_PER_SUB
ZUNROLL = 4


def _prep_body(x_ref, w1a_ref, w1b_ref, atab_ref, betab_ref):
    x = x_ref[...]
    atab_ref[...] = jnp.dot(x, w1a_ref[...], preferred_element_type=jnp.float32)
    b = jnp.dot(x, w1b_ref[...], preferred_element_type=jnp.float32)
    betab_ref[...] = jnp.concatenate([b, x], axis=1)


def _prep_tables(emb, w1a, w1b):
    bn = 2000
    return pl.pallas_call(
        _prep_body,
        grid=(N_NODES // bn,),
        in_specs=[pl.BlockSpec((bn, D), lambda i: (i, 0)),
                  pl.BlockSpec((D, D), lambda i: (0, 0)),
                  pl.BlockSpec((D, D), lambda i: (0, 0))],
        out_specs=[pl.BlockSpec((bn, D), lambda i: (i, 0)),
                   pl.BlockSpec((bn, 2 * D), lambda i: (i, 0))],
        out_shape=[jax.ShapeDtypeStruct((N_NODES, D), jnp.float32),
                   jax.ShapeDtypeStruct((N_NODES, 2 * D), jnp.float32)],
    )(emb, w1a, w1b)


def _mid_body(p_ref, w1a_ref, w1b_ref, emb_ref, atab_ref, betab_ref):
    x = p_ref[0] + p_ref[1]
    emb_ref[...] = x
    atab_ref[...] = jnp.dot(x, w1a_ref[...], preferred_element_type=jnp.float32)
    b = jnp.dot(x, w1b_ref[...], preferred_element_type=jnp.float32)
    betab_ref[...] = jnp.concatenate([b, x], axis=1)


def _mid_tables(partials, w1a, w1b):
    bn = 2000
    return pl.pallas_call(
        _mid_body,
        grid=(N_NODES // bn,),
        in_specs=[pl.BlockSpec((2, bn, D), lambda i: (0, i, 0)),
                  pl.BlockSpec((D, D), lambda i: (0, 0)),
                  pl.BlockSpec((D, D), lambda i: (0, 0))],
        out_specs=[pl.BlockSpec((bn, D), lambda i: (i, 0)),
                   pl.BlockSpec((bn, D), lambda i: (i, 0)),
                   pl.BlockSpec((bn, 2 * D), lambda i: (i, 0))],
        out_shape=[jax.ShapeDtypeStruct((N_NODES, D), jnp.float32),
                   jax.ShapeDtypeStruct((N_NODES, D), jnp.float32),
                   jax.ShapeDtypeStruct((N_NODES, 2 * D), jnp.float32)],
    )(partials, w1a, w1b)


def _final_body(e0_ref, e1_ref, p_ref, out_ref):
    out_ref[...] = (e0_ref[...] + e1_ref[...] + p_ref[0] + p_ref[1]) * (
        1.0 / (N_LAYERS + 1))


def _final_mean(emb0, emb1, partials2):
    bn = 2000
    return pl.pallas_call(
        _final_body,
        grid=(N_NODES // bn,),
        in_specs=[pl.BlockSpec((bn, D), lambda i: (i, 0)),
                  pl.BlockSpec((bn, D), lambda i: (i, 0)),
                  pl.BlockSpec((2, bn, D), lambda i: (0, i, 0))],
        out_specs=pl.BlockSpec((bn, D), lambda i: (i, 0)),
        out_shape=jax.ShapeDtypeStruct((N_NODES, D), jnp.float32),
    )(emb0, emb1, partials2)


def _make_edge_kernel(scale):
    mesh = plsc.VectorSubcoreMesh(core_axis_name="c", subcore_axis_name="s")

    @functools.partial(
        pl.kernel, mesh=mesh,
        compiler_params=pltpu.CompilerParams(needs_layout_passes=False,
                                             use_tc_tiling_on_sc=False),
        out_type=(jax.ShapeDtypeStruct((2, N_NODES, D), jnp.float32),
                  jax.ShapeDtypeStruct((EP,), jnp.float32)),
        scratch_types=[
            pltpu.VMEM_SHARED((N_NODES, D), jnp.float32),
            pltpu.VMEM((4, K), jnp.int32),
            pltpu.VMEM((4, K), jnp.int32),
            pltpu.VMEM((4, K), jnp.float32),
            pltpu.VMEM((2, K), jnp.float32),
            pltpu.VMEM((2, K, D), jnp.float32),
            pltpu.VMEM((2, K, 2 * D), jnp.float32),
            pltpu.VMEM((2, K, D), jnp.float32),
            pltpu.VMEM((D,), jnp.float32),
            pltpu.SemaphoreType.DMA,
            pltpu.SemaphoreType.DMA,
            pltpu.SemaphoreType.DMA,
            pltpu.SemaphoreType.DMA,
            pltpu.SemaphoreType.DMA,
            pltpu.SemaphoreType.DMA,
            pltpu.SemaphoreType.DMA,
            pltpu.SemaphoreType.DMA,
        ],
    )
    def edge_kernel(atab, betab, rows_hbm, cols_hbm, gv_hbm, w2_hbm, zeros_hbm,
                    outp, gv_out, acc, rows_v, cols_v, gv_v, gvn_v, S, T, M,
                    w2_v, sg0, sg1, si0, si1, sm0, sm1, sn0, sn1):
        cid = lax.axis_index("c")
        sid = lax.axis_index("s")
        w = cid * 16 + sid
        sem_g = (sg0, sg1)
        sem_i = (si0, si1)
        sem_m = (sm0, sm1)
        sem_n = (sn0, sn1)
        base0 = w * (NCH * K)

        pltpu.sync_copy(w2_hbm, w2_v)

        @pl.when(sid < 15)
        def _():
            pltpu.sync_copy(zeros_hbm,
                            acc.at[pl.ds(sid * ROWS_PER_SUB, ROWS_PER_SUB)])

        @pl.when(sid == 15)
        def _():
            pltpu.sync_copy(zeros_hbm.at[pl.ds(0, ROWS_LAST)],
                            acc.at[pl.ds(15 * ROWS_PER_SUB, ROWS_LAST)])

        plsc.subcore_barrier()

        for b in range(2):
            pltpu.sync_copy(rows_hbm.at[pl.ds(base0 + b * K, K)], rows_v.at[b])
            pltpu.sync_copy(cols_hbm.at[pl.ds(base0 + b * K, K)], cols_v.at[b])
            pltpu.sync_copy(gv_hbm.at[pl.ds(base0 + b * K, K)], gv_v.at[b])
            pltpu.async_copy(atab.at[rows_v.at[b]], S.at[b], sem_g[b])
            pltpu.async_copy(betab.at[cols_v.at[b]], T.at[b], sem_g[b])

        erows = [lax.iota(jnp.int32, 16) + g * 16 for g in range(NGRP)]

        def quad_body(q, _carry):
            for j in range(4):
                p = j % 2
                ck = 4 * q + j
                base = base0 + ck * K
                Sb, Tb, Mb = S.at[p], T.at[p], M.at[p]
                pltpu.make_async_copy(atab.at[rows_v.at[j]], Sb,
                                      sem_g[p]).wait()
                pltpu.make_async_copy(betab.at[cols_v.at[j]], Tb,
                                      sem_g[p]).wait()
                @pl.when(ck >= 2)
                def _():
                    pltpu.make_async_copy(
                        Mb, acc.at[rows_v.at[(j + 2) % 4]], sem_m[p]).wait()
                    pltpu.make_async_copy(
                        gvn_v.at[p], gv_out.at[pl.ds(base, K)],
                        sem_n[p]).wait()
                @pl.when((q < NCH // 4 - 1) if j >= 2 else (q >= 0))
                def _():
                    nb = base + 2 * K
                    sl = (j + 2) % 4
                    pltpu.async_copy(rows_hbm.at[pl.ds(nb, K)],
                                     rows_v.at[sl], sem_i[p])
                    pltpu.async_copy(cols_hbm.at[pl.ds(nb, K)],
                                     cols_v.at[sl], sem_i[p])
                    pltpu.async_copy(gv_hbm.at[pl.ds(nb, K)],
                                     gv_v.at[sl], sem_i[p])

                def zbody(jj, accs):
                    out = list(accs)
                    for dd in range(ZUNROLL):
                        d = jj * ZUNROLL + dd
                        dsplat = jnp.broadcast_to(d, (16,))
                        w2d = plsc.load_gather(w2_v, [dsplat])
                        for g in range(NGRP):
                            a = plsc.load_gather(Sb, [erows[g], dsplat])
                            t = plsc.load_gather(Tb, [erows[g], dsplat])
                            out[g] = out[g] + jnp.maximum(a + t, 0.0) * w2d
                    return tuple(out)

                z0 = tuple(jnp.zeros((16,), jnp.float32) for _ in range(NGRP))
                zs = lax.fori_loop(0, D // ZUNROLL, zbody, z0)

                gvb = gv_v.at[j]
                gvnb = gvn_v.at[p]
                gvns = []
                for g in range(NGRP):
                    sig = 1.0 / (1.0 + jnp.exp(-zs[g]))
                    gvn = gvb[pl.ds(g * 16, 16)] * jnp.exp(sig * (-scale))
                    gvnb[pl.ds(g * 16, 16)] = gvn
                    gvns.append(gvn)

                def mbody(jj, _c2):
                    for dd in range(ZUNROLL):
                        d = jj * ZUNROLL + dd
                        dsplat = jnp.broadcast_to(d, (16,))
                        tsplat = dsplat + D
                        for g in range(NGRP):
                            t = plsc.load_gather(Tb, [erows[g], tsplat])
                            plsc.store_scatter(Mb, [erows[g], dsplat],
                                               t * gvns[g])
                    return 0

                lax.fori_loop(0, D // ZUNROLL, mbody, 0)

                pltpu.async_copy(Mb, acc.at[rows_v.at[j]], sem_m[p], add=True)
                pltpu.async_copy(gvnb, gv_out.at[pl.ds(base, K)], sem_n[p])

                @pl.when((q < NCH // 4 - 1) if j >= 2 else (q >= 0))
                def _():
                    sl = (j + 2) % 4
                    pltpu.make_async_copy(rows_hbm.at[pl.ds(base, K)],
                                          rows_v.at[sl], sem_i[p]).wait()
                    pltpu.make_async_copy(cols_hbm.at[pl.ds(base, K)],
                                          cols_v.at[sl], sem_i[p]).wait()
                    pltpu.make_async_copy(gv_hbm.at[pl.ds(base, K)],
                                          gv_v.at[sl], sem_i[p]).wait()
                    pltpu.async_copy(atab.at[rows_v.at[sl]], Sb, sem_g[p])
                    pltpu.async_copy(betab.at[cols_v.at[sl]], Tb, sem_g[p])
            return 0

        lax.fori_loop(0, NCH // 4, quad_body, 0)

        for p in range(2):
            ck = NCH - 2 + p
            pltpu.make_async_copy(M.at[p], acc.at[rows_v.at[ck % 4]],
                                  sem_m[p]).wait()
            pltpu.make_async_copy(gvn_v.at[p],
                                  gv_out.at[pl.ds(base0 + ck * K, K)],
                                  sem_n[p]).wait()
        plsc.subcore_barrier()

        @pl.when(sid < 15)
        def _():
            pltpu.sync_copy(acc.at[pl.ds(sid * ROWS_PER_SUB, ROWS_PER_SUB)],
                            outp.at[cid, pl.ds(sid * ROWS_PER_SUB,
                                               ROWS_PER_SUB)])

        @pl.when(sid == 15)
        def _():
            pltpu.sync_copy(acc.at[pl.ds(15 * ROWS_PER_SUB, ROWS_LAST)],
                            outp.at[cid, pl.ds(15 * ROWS_PER_SUB, ROWS_LAST)])

    return edge_kernel


_edge_kernels = [_make_edge_kernel(float(layer + 1)) for layer in range(N_LAYERS)]


def kernel(user_emb, item_emb, g_values, W1, W2, g_row, g_col):
    emb0 = jnp.concatenate([user_emb, item_emb], axis=0)
    w1a, w1b = W1[:D], W1[D:]
    w2 = jnp.reshape(W2, (D,))
    pad = EP - E
    rows = jnp.concatenate([g_row, jnp.zeros((pad,), jnp.int32)])
    cols = jnp.concatenate([g_col, jnp.zeros((pad,), jnp.int32)])
    gv = jnp.concatenate([g_values, jnp.zeros((pad,), jnp.float32)])
    zeros = jnp.zeros((ROWS_PER_SUB, D), jnp.float32)

    atab, betab = _prep_tables(emb0, w1a, w1b)
    partials1, gv = _edge_kernels[0](atab, betab, rows, cols, gv, w2, zeros)
    emb1, atab, betab = _mid_tables(partials1, w1a, w1b)
    partials2, gv = _edge_kernels[1](atab, betab, rows, cols, gv, w2, zeros)
    return _final_mean(emb0, emb1, partials2)


# speedup vs baseline: 1.5826x; 1.3193x over previous
import functools

import jax
import jax.numpy as jnp
from jax import lax
from jax.experimental import pallas as pl
from jax.experimental.pallas import tpu as pltpu
from jax.experimental.pallas import tpu_sc as plsc

N_NODES = 10000
D = 128
E = 320000
N_LAYERS = 2

NW = 32
K = 48
NCH = 212
EP = NW * NCH * K
NGRP = K // 16
ROWS_PER_SUB = 632
ROWS_LAST = N_NODES - 15 * ROWS_PER_SUB
ZUNROLL = 4


def _prep_body(x_ref, w1a_ref, w1b_ref, atab_ref, betab_ref):
    x = x_ref[...]
    atab_ref[...] = jnp.dot(x, w1a_ref[...], preferred_element_type=jnp.float32)
    b = jnp.dot(x, w1b_ref[...], preferred_element_type=jnp.float32)
    betab_ref[...] = jnp.concatenate([b, x], axis=1)


def _prep_tables(emb, w1a, w1b):
    bn = 2000
    return pl.pallas_call(
        _prep_body,
        grid=(N_NODES // bn,),
        in_specs=[pl.BlockSpec((bn, D), lambda i: (i, 0)),
                  pl.BlockSpec((D, D), lambda i: (0, 0)),
                  pl.BlockSpec((D, D), lambda i: (0, 0))],
        out_specs=[pl.BlockSpec((bn, D), lambda i: (i, 0)),
                   pl.BlockSpec((bn, 2 * D), lambda i: (i, 0))],
        out_shape=[jax.ShapeDtypeStruct((N_NODES, D), jnp.float32),
                   jax.ShapeDtypeStruct((N_NODES, 2 * D), jnp.float32)],
    )(emb, w1a, w1b)


def _mid_body(p_ref, w1a_ref, w1b_ref, emb_ref, atab_ref, betab_ref):
    x = p_ref[0] + p_ref[1]
    emb_ref[...] = x
    atab_ref[...] = jnp.dot(x, w1a_ref[...], preferred_element_type=jnp.float32)
    b = jnp.dot(x, w1b_ref[...], preferred_element_type=jnp.float32)
    betab_ref[...] = jnp.concatenate([b, x], axis=1)


def _mid_tables(partials, w1a, w1b):
    bn = 2000
    return pl.pallas_call(
        _mid_body,
        grid=(N_NODES // bn,),
        in_specs=[pl.BlockSpec((2, bn, D), lambda i: (0, i, 0)),
                  pl.BlockSpec((D, D), lambda i: (0, 0)),
                  pl.BlockSpec((D, D), lambda i: (0, 0))],
        out_specs=[pl.BlockSpec((bn, D), lambda i: (i, 0)),
                   pl.BlockSpec((bn, D), lambda i: (i, 0)),
                   pl.BlockSpec((bn, 2 * D), lambda i: (i, 0))],
        out_shape=[jax.ShapeDtypeStruct((N_NODES, D), jnp.float32),
                   jax.ShapeDtypeStruct((N_NODES, D), jnp.float32),
                   jax.ShapeDtypeStruct((N_NODES, 2 * D), jnp.float32)],
    )(partials, w1a, w1b)


def _final_body(e0_ref, e1_ref, p_ref, out_ref):
    out_ref[...] = (e0_ref[...] + e1_ref[...] + p_ref[0] + p_ref[1]) * (
        1.0 / (N_LAYERS + 1))


def _final_mean(emb0, emb1, partials2):
    bn = 2000
    return pl.pallas_call(
        _final_body,
        grid=(N_NODES // bn,),
        in_specs=[pl.BlockSpec((bn, D), lambda i: (i, 0)),
                  pl.BlockSpec((bn, D), lambda i: (i, 0)),
                  pl.BlockSpec((2, bn, D), lambda i: (0, i, 0))],
        out_specs=pl.BlockSpec((bn, D), lambda i: (i, 0)),
        out_shape=jax.ShapeDtypeStruct((N_NODES, D), jnp.float32),
    )(emb0, emb1, partials2)


def _make_edge_kernel(scale):
    mesh = plsc.VectorSubcoreMesh(core_axis_name="c", subcore_axis_name="s")

    @functools.partial(
        pl.kernel, mesh=mesh,
        compiler_params=pltpu.CompilerParams(needs_layout_passes=False,
                                             use_tc_tiling_on_sc=False),
        out_type=(jax.ShapeDtypeStruct((2, N_NODES, D), jnp.float32),
                  jax.ShapeDtypeStruct((EP,), jnp.float32)),
        scratch_types=[
            pltpu.VMEM_SHARED((N_NODES, D), jnp.float32),
            pltpu.VMEM((4, K), jnp.int32),
            pltpu.VMEM((4, K), jnp.int32),
            pltpu.VMEM((4, K), jnp.float32),
            pltpu.VMEM((2, K), jnp.float32),
            pltpu.VMEM((2, K, D), jnp.float32),
            pltpu.VMEM((2, K, 2 * D), jnp.float32),
            pltpu.VMEM((2, K, D), jnp.float32),
            pltpu.VMEM((D,), jnp.float32),
            pltpu.SemaphoreType.DMA,
            pltpu.SemaphoreType.DMA,
            pltpu.SemaphoreType.DMA,
            pltpu.SemaphoreType.DMA,
            pltpu.SemaphoreType.DMA,
            pltpu.SemaphoreType.DMA,
            pltpu.SemaphoreType.DMA,
            pltpu.SemaphoreType.DMA,
        ],
    )
    def edge_kernel(atab, betab, rows_hbm, cols_hbm, gv_hbm, w2_hbm, zeros_hbm,
                    outp, gv_out, acc, rows_v, cols_v, gv_v, gvn_v, S, T, M,
                    w2_v, sg0, sg1, si0, si1, sm0, sm1, sn0, sn1):
        cid = lax.axis_index("c")
        sid = lax.axis_index("s")
        w = cid * 16 + sid
        sem_g = (sg0, sg1)
        sem_i = (si0, si1)
        sem_m = (sm0, sm1)
        sem_n = (sn0, sn1)
        base0 = w * (NCH * K)

        pltpu.sync_copy(w2_hbm, w2_v)

        @pl.when(sid < 15)
        def _():
            pltpu.sync_copy(zeros_hbm,
                            acc.at[pl.ds(sid * ROWS_PER_SUB, ROWS_PER_SUB)])

        @pl.when(sid == 15)
        def _():
            pltpu.sync_copy(zeros_hbm.at[pl.ds(0, ROWS_LAST)],
                            acc.at[pl.ds(15 * ROWS_PER_SUB, ROWS_LAST)])

        plsc.subcore_barrier()

        for b in range(2):
            pltpu.sync_copy(rows_hbm.at[pl.ds(base0 + b * K, K)], rows_v.at[b])
            pltpu.sync_copy(cols_hbm.at[pl.ds(base0 + b * K, K)], cols_v.at[b])
            pltpu.sync_copy(gv_hbm.at[pl.ds(base0 + b * K, K)], gv_v.at[b])
            pltpu.async_copy(atab.at[rows_v.at[b]], S.at[b], sem_g[b])
            pltpu.async_copy(betab.at[cols_v.at[b]], T.at[b], sem_g[b])

        erows = [lax.iota(jnp.int32, 16) + g * 16 for g in range(NGRP)]

        def quad_body(q, _carry):
            for j in range(4):
                p = j % 2
                ck = 4 * q + j
                base = base0 + ck * K
                Sb, Tb, Mb = S.at[p], T.at[p], M.at[p]
                pltpu.make_async_copy(atab.at[rows_v.at[j]], Sb,
                                      sem_g[p]).wait()
                pltpu.make_async_copy(betab.at[cols_v.at[j]], Tb,
                                      sem_g[p]).wait()
                @pl.when(ck >= 2)
                def _():
                    pltpu.make_async_copy(
                        Mb, acc.at[rows_v.at[(j + 2) % 4]], sem_m[p]).wait()
                    pltpu.make_async_copy(
                        gvn_v.at[p], gv_out.at[pl.ds(base, K)],
                        sem_n[p]).wait()
                @pl.when((q < NCH // 4 - 1) if j >= 2 else (q >= 0))
                def _():
                    nb = base + 2 * K
                    sl = (j + 2) % 4
                    pltpu.async_copy(rows_hbm.at[pl.ds(nb, K)],
                                     rows_v.at[sl], sem_i[p])
                    pltpu.async_copy(cols_hbm.at[pl.ds(nb, K)],
                                     cols_v.at[sl], sem_i[p])
                    pltpu.async_copy(gv_hbm.at[pl.ds(nb, K)],
                                     gv_v.at[sl], sem_i[p])

                def zbody(jj, accs):
                    out = list(accs)
                    for dd in range(ZUNROLL):
                        d = jj * ZUNROLL + dd
                        dsplat = jnp.broadcast_to(d, (16,))
                        w2d = plsc.load_gather(w2_v, [dsplat])
                        for g in range(NGRP):
                            a = plsc.load_gather(Sb, [erows[g], dsplat])
                            t = plsc.load_gather(Tb, [erows[g], dsplat])
                            out[g] = out[g] + jnp.maximum(a + t, 0.0) * w2d
                    return tuple(out)

                z0 = tuple(jnp.zeros((16,), jnp.float32) for _ in range(NGRP))
                zs = lax.fori_loop(0, D // ZUNROLL, zbody, z0)

                gvb = gv_v.at[j]
                gvnb = gvn_v.at[p]
                gvns = []
                for g in range(NGRP):
                    sig = 1.0 / (1.0 + jnp.exp(-zs[g]))
                    gvn = gvb[pl.ds(g * 16, 16)] * jnp.exp(sig * (-scale))
                    gvnb[pl.ds(g * 16, 16)] = gvn
                    gvns.append(gvn)

                @plsc.parallel_loop(0, D // ZUNROLL, unroll=2)
                def _(jj):
                    for dd in range(ZUNROLL):
                        d = jj * ZUNROLL + dd
                        dsplat = jnp.broadcast_to(d, (16,))
                        tsplat = dsplat + D
                        for g in range(NGRP):
                            t = plsc.load_gather(Tb, [erows[g], tsplat])
                            plsc.store_scatter(Mb, [erows[g], dsplat],
                                               t * gvns[g])

                pltpu.async_copy(Mb, acc.at[rows_v.at[j]], sem_m[p], add=True)
                pltpu.async_copy(gvnb, gv_out.at[pl.ds(base, K)], sem_n[p])

                @pl.when((q < NCH // 4 - 1) if j >= 2 else (q >= 0))
                def _():
                    sl = (j + 2) % 4
                    pltpu.make_async_copy(rows_hbm.at[pl.ds(base, K)],
                                          rows_v.at[sl], sem_i[p]).wait()
                    pltpu.make_async_copy(cols_hbm.at[pl.ds(base, K)],
                                          cols_v.at[sl], sem_i[p]).wait()
                    pltpu.make_async_copy(gv_hbm.at[pl.ds(base, K)],
                                          gv_v.at[sl], sem_i[p]).wait()
                    pltpu.async_copy(atab.at[rows_v.at[sl]], Sb, sem_g[p])
                    pltpu.async_copy(betab.at[cols_v.at[sl]], Tb, sem_g[p])
            return 0

        lax.fori_loop(0, NCH // 4, quad_body, 0)

        for p in range(2):
            ck = NCH - 2 + p
            pltpu.make_async_copy(M.at[p], acc.at[rows_v.at[ck % 4]],
                                  sem_m[p]).wait()
            pltpu.make_async_copy(gvn_v.at[p],
                                  gv_out.at[pl.ds(base0 + ck * K, K)],
                                  sem_n[p]).wait()
        plsc.subcore_barrier()

        @pl.when(sid < 15)
        def _():
            pltpu.sync_copy(acc.at[pl.ds(sid * ROWS_PER_SUB, ROWS_PER_SUB)],
                            outp.at[cid, pl.ds(sid * ROWS_PER_SUB,
                                               ROWS_PER_SUB)])

        @pl.when(sid == 15)
        def _():
            pltpu.sync_copy(acc.at[pl.ds(15 * ROWS_PER_SUB, ROWS_LAST)],
                            outp.at[cid, pl.ds(15 * ROWS_PER_SUB, ROWS_LAST)])

    return edge_kernel


_edge_kernels = [_make_edge_kernel(float(layer + 1)) for layer in range(N_LAYERS)]


def kernel(user_emb, item_emb, g_values, W1, W2, g_row, g_col):
    emb0 = jnp.concatenate([user_emb, item_emb], axis=0)
    w1a, w1b = W1[:D], W1[D:]
    w2 = jnp.reshape(W2, (D,))
    pad = EP - E
    rows = jnp.concatenate([g_row, jnp.zeros((pad,), jnp.int32)])
    cols = jnp.concatenate([g_col, jnp.zeros((pad,), jnp.int32)])
    gv = jnp.concatenate([g_values, jnp.zeros((pad,), jnp.float32)])
    zeros = jnp.zeros((ROWS_PER_SUB, D), jnp.float32)

    atab, betab = _prep_tables(emb0, w1a, w1b)
    partials1, gv = _edge_kernels[0](atab, betab, rows, cols, gv, w2, zeros)
    emb1, atab, betab = _mid_tables(partials1, w1a, w1b)
    partials2, gv = _edge_kernels[1](atab, betab, rows, cols, gv, w2, zeros)
    return _final_mean(emb0, emb1, partials2)


# speedup vs baseline: 2.0747x; 1.3110x over previous
import functools

import jax
import jax.numpy as jnp
from jax import lax
from jax.experimental import pallas as pl
from jax.experimental.pallas import tpu as pltpu
from jax.experimental.pallas import tpu_sc as plsc

N_NODES = 10000
D = 128
E = 320000
N_LAYERS = 2

NW = 32
K = 64
NCH = 160
EP = NW * NCH * K
NGRP = K // 16
ROWS_PER_SUB = 632
ROWS_LAST = N_NODES - 15 * ROWS_PER_SUB
WPR = D // 2

_ILV = plsc.PackFormat.INTERLEAVED


def _prep_body(x_ref, w1a_ref, w1b_ref, atab_ref, betab_ref):
    x = x_ref[...]
    a = jnp.dot(x, w1a_ref[...], preferred_element_type=jnp.float32)
    atab_ref[...] = a.astype(jnp.bfloat16)
    b = jnp.dot(x, w1b_ref[...], preferred_element_type=jnp.float32)
    betab_ref[...] = jnp.concatenate([b, x], axis=1).astype(jnp.bfloat16)


def _prep_tables(emb, w1a, w1b):
    bn = 2000
    return pl.pallas_call(
        _prep_body,
        grid=(N_NODES // bn,),
        in_specs=[pl.BlockSpec((bn, D), lambda i: (i, 0)),
                  pl.BlockSpec((D, D), lambda i: (0, 0)),
                  pl.BlockSpec((D, D), lambda i: (0, 0))],
        out_specs=[pl.BlockSpec((bn, D), lambda i: (i, 0)),
                   pl.BlockSpec((bn, 2 * D), lambda i: (i, 0))],
        out_shape=[jax.ShapeDtypeStruct((N_NODES, D), jnp.bfloat16),
                   jax.ShapeDtypeStruct((N_NODES, 2 * D), jnp.bfloat16)],
    )(emb, w1a, w1b)


def _mid_body(p_ref, w1a_ref, w1b_ref, emb_ref, atab_ref, betab_ref):
    x = p_ref[0] + p_ref[1]
    emb_ref[...] = x
    a = jnp.dot(x, w1a_ref[...], preferred_element_type=jnp.float32)
    atab_ref[...] = a.astype(jnp.bfloat16)
    b = jnp.dot(x, w1b_ref[...], preferred_element_type=jnp.float32)
    betab_ref[...] = jnp.concatenate([b, x], axis=1).astype(jnp.bfloat16)


def _mid_tables(partials, w1a, w1b):
    bn = 2000
    return pl.pallas_call(
        _mid_body,
        grid=(N_NODES // bn,),
        in_specs=[pl.BlockSpec((2, bn, D), lambda i: (0, i, 0)),
                  pl.BlockSpec((D, D), lambda i: (0, 0)),
                  pl.BlockSpec((D, D), lambda i: (0, 0))],
        out_specs=[pl.BlockSpec((bn, D), lambda i: (i, 0)),
                   pl.BlockSpec((bn, D), lambda i: (i, 0)),
                   pl.BlockSpec((bn, 2 * D), lambda i: (i, 0))],
        out_shape=[jax.ShapeDtypeStruct((N_NODES, D), jnp.float32),
                   jax.ShapeDtypeStruct((N_NODES, D), jnp.bfloat16),
                   jax.ShapeDtypeStruct((N_NODES, 2 * D), jnp.bfloat16)],
    )(partials, w1a, w1b)


def _final_body(e0_ref, e1_ref, p_ref, out_ref):
    out_ref[...] = (e0_ref[...] + e1_ref[...] + p_ref[0] + p_ref[1]) * (
        1.0 / (N_LAYERS + 1))


def _final_mean(emb0, emb1, partials2):
    bn = 2000
    return pl.pallas_call(
        _final_body,
        grid=(N_NODES // bn,),
        in_specs=[pl.BlockSpec((bn, D), lambda i: (i, 0)),
                  pl.BlockSpec((bn, D), lambda i: (i, 0)),
                  pl.BlockSpec((2, bn, D), lambda i: (0, i, 0))],
        out_specs=pl.BlockSpec((bn, D), lambda i: (i, 0)),
        out_shape=jax.ShapeDtypeStruct((N_NODES, D), jnp.float32),
    )(emb0, emb1, partials2)


def _pack_words(x_bf16):
    n, m = x_bf16.shape
    return jax.lax.bitcast_convert_type(
        x_bf16.reshape(n, m // 2, 2), jnp.int32)


def _make_edge_kernel(scale):
    mesh = plsc.VectorSubcoreMesh(core_axis_name="c", subcore_axis_name="s")

    @functools.partial(
        pl.kernel, mesh=mesh,
        compiler_params=pltpu.CompilerParams(needs_layout_passes=False,
                                             use_tc_tiling_on_sc=False),
        out_type=(jax.ShapeDtypeStruct((2, N_NODES, D), jnp.float32),
                  jax.ShapeDtypeStruct((EP,), jnp.float32)),
        scratch_types=[
            pltpu.VMEM_SHARED((N_NODES, D), jnp.float32),
            pltpu.VMEM((4, K), jnp.int32),
            pltpu.VMEM((4, K), jnp.int32),
            pltpu.VMEM((4, K), jnp.float32),
            pltpu.VMEM((2, K), jnp.float32),
            pltpu.VMEM((2, K, WPR), jnp.int32),
            pltpu.VMEM((2, K, 2 * WPR), jnp.int32),
            pltpu.VMEM((2, K, D), jnp.float32),
            pltpu.VMEM((WPR,), jnp.int32),
            pltpu.SemaphoreType.DMA,
            pltpu.SemaphoreType.DMA,
            pltpu.SemaphoreType.DMA,
            pltpu.SemaphoreType.DMA,
            pltpu.SemaphoreType.DMA,
            pltpu.SemaphoreType.DMA,
            pltpu.SemaphoreType.DMA,
            pltpu.SemaphoreType.DMA,
        ],
    )
    def edge_kernel(atab, betab, rows_hbm, cols_hbm, gv_hbm, w2_hbm, zeros_hbm,
                    outp, gv_out, acc, rows_v, cols_v, gv_v, gvn_v, S, T, M,
                    w2_v, sg0, sg1, si0, si1, sm0, sm1, sn0, sn1):
        cid = lax.axis_index("c")
        sid = lax.axis_index("s")
        w = cid * 16 + sid
        sem_g = (sg0, sg1)
        sem_i = (si0, si1)
        sem_m = (sm0, sm1)
        sem_n = (sn0, sn1)
        base0 = w * (NCH * K)

        pltpu.sync_copy(w2_hbm, w2_v)

        @pl.when(sid < 15)
        def _():
            pltpu.sync_copy(zeros_hbm,
                            acc.at[pl.ds(sid * ROWS_PER_SUB, ROWS_PER_SUB)])

        @pl.when(sid == 15)
        def _():
            pltpu.sync_copy(zeros_hbm.at[pl.ds(0, ROWS_LAST)],
                            acc.at[pl.ds(15 * ROWS_PER_SUB, ROWS_LAST)])

        plsc.subcore_barrier()

        for b in range(2):
            pltpu.sync_copy(rows_hbm.at[pl.ds(base0 + b * K, K)], rows_v.at[b])
            pltpu.sync_copy(cols_hbm.at[pl.ds(base0 + b * K, K)], cols_v.at[b])
            pltpu.sync_copy(gv_hbm.at[pl.ds(base0 + b * K, K)], gv_v.at[b])
            pltpu.async_copy(atab.at[rows_v.at[b]], S.at[b], sem_g[b])
            pltpu.async_copy(betab.at[cols_v.at[b]], T.at[b], sem_g[b])

        erows = [lax.iota(jnp.int32, 16) + g * 16 for g in range(NGRP)]

        def quad_body(q, _carry):
            for j in range(4):
                p = j % 2
                ck = 4 * q + j
                base = base0 + ck * K
                Sb, Tb, Mb = S.at[p], T.at[p], M.at[p]
                pltpu.make_async_copy(atab.at[rows_v.at[j]], Sb,
                                      sem_g[p]).wait()
                pltpu.make_async_copy(betab.at[cols_v.at[j]], Tb,
                                      sem_g[p]).wait()
                @pl.when(ck >= 2)
                def _():
                    pltpu.make_async_copy(
                        Mb, acc.at[rows_v.at[(j + 2) % 4]], sem_m[p]).wait()
                    pltpu.make_async_copy(
                        gvn_v.at[p], gv_out.at[pl.ds(base, K)],
                        sem_n[p]).wait()
                @pl.when((q < NCH // 4 - 1) if j >= 2 else (q >= 0))
                def _():
                    nb = base + 2 * K
                    sl = (j + 2) % 4
                    pltpu.async_copy(rows_hbm.at[pl.ds(nb, K)],
                                     rows_v.at[sl], sem_i[p])
                    pltpu.async_copy(cols_hbm.at[pl.ds(nb, K)],
                                     cols_v.at[sl], sem_i[p])
                    pltpu.async_copy(gv_hbm.at[pl.ds(nb, K)],
                                     gv_v.at[sl], sem_i[p])

                def zbody(jj, accs):
                    out = list(accs)
                    for dd in range(2):
                        wd = jj * 2 + dd
                        wsplat = jnp.broadcast_to(wd, (16,))
                        w2p = plsc.bitcast(plsc.load_gather(w2_v, [wsplat]),
                                           jnp.bfloat16)
                        w2e, w2o = plsc.unpack(w2p, format=_ILV)
                        for g in range(NGRP):
                            ap = plsc.bitcast(
                                plsc.load_gather(Sb, [erows[g], wsplat]),
                                jnp.bfloat16)
                            bp = plsc.bitcast(
                                plsc.load_gather(Tb, [erows[g], wsplat]),
                                jnp.bfloat16)
                            ae, ao = plsc.unpack(ap, format=_ILV)
                            be, bo = plsc.unpack(bp, format=_ILV)
                            out[g] = (out[g]
                                      + jnp.maximum(ae + be, 0.0) * w2e
                                      + jnp.maximum(ao + bo, 0.0) * w2o)
                    return tuple(out)

                z0 = tuple(jnp.zeros((16,), jnp.float32) for _ in range(NGRP))
                zs = lax.fori_loop(0, WPR // 2, zbody, z0)

                gvb = gv_v.at[j]
                gvnb = gvn_v.at[p]
                gvns = []
                for g in range(NGRP):
                    sig = 1.0 / (1.0 + jnp.exp(-zs[g]))
                    gvn = gvb[pl.ds(g * 16, 16)] * jnp.exp(sig * (-scale))
                    gvnb[pl.ds(g * 16, 16)] = gvn
                    gvns.append(gvn)

                @plsc.parallel_loop(0, WPR, unroll=2)
                def _(wd):
                    wsplat = jnp.broadcast_to(wd, (16,))
                    esplat = wsplat * 2
                    osplat = esplat + 1
                    for g in range(NGRP):
                        ep = plsc.bitcast(
                            plsc.load_gather(Tb, [erows[g], wsplat + WPR]),
                            jnp.bfloat16)
                        ee, eo = plsc.unpack(ep, format=_ILV)
                        plsc.store_scatter(Mb, [erows[g], esplat],
                                           ee * gvns[g])
                        plsc.store_scatter(Mb, [erows[g], osplat],
                                           eo * gvns[g])

                pltpu.async_copy(Mb, acc.at[rows_v.at[j]], sem_m[p], add=True)
                pltpu.async_copy(gvnb, gv_out.at[pl.ds(base, K)], sem_n[p])

                @pl.when((q < NCH // 4 - 1) if j >= 2 else (q >= 0))
                def _():
                    sl = (j + 2) % 4
                    pltpu.make_async_copy(rows_hbm.at[pl.ds(base, K)],
                                          rows_v.at[sl], sem_i[p]).wait()
                    pltpu.make_async_copy(cols_hbm.at[pl.ds(base, K)],
                                          cols_v.at[sl], sem_i[p]).wait()
                    pltpu.make_async_copy(gv_hbm.at[pl.ds(base, K)],
                                          gv_v.at[sl], sem_i[p]).wait()
                    pltpu.async_copy(atab.at[rows_v.at[sl]], Sb, sem_g[p])
                    pltpu.async_copy(betab.at[cols_v.at[sl]], Tb, sem_g[p])
            return 0

        lax.fori_loop(0, NCH // 4, quad_body, 0)

        for p in range(2):
            ck = NCH - 2 + p
            pltpu.make_async_copy(M.at[p], acc.at[rows_v.at[ck % 4]],
                                  sem_m[p]).wait()
            pltpu.make_async_copy(gvn_v.at[p],
                                  gv_out.at[pl.ds(base0 + ck * K, K)],
                                  sem_n[p]).wait()
        plsc.subcore_barrier()

        @pl.when(sid < 15)
        def _():
            pltpu.sync_copy(acc.at[pl.ds(sid * ROWS_PER_SUB, ROWS_PER_SUB)],
                            outp.at[cid, pl.ds(sid * ROWS_PER_SUB,
                                               ROWS_PER_SUB)])

        @pl.when(sid == 15)
        def _():
            pltpu.sync_copy(acc.at[pl.ds(15 * ROWS_PER_SUB, ROWS_LAST)],
                            outp.at[cid, pl.ds(15 * ROWS_PER_SUB, ROWS_LAST)])

    return edge_kernel


_edge_kernels = [_make_edge_kernel(float(layer + 1)) for layer in range(N_LAYERS)]


def kernel(user_emb, item_emb, g_values, W1, W2, g_row, g_col):
    emb0 = jnp.concatenate([user_emb, item_emb], axis=0)
    w1a, w1b = W1[:D], W1[D:]
    w2p = _pack_words(jnp.reshape(W2, (1, D)).astype(jnp.bfloat16))[0]
    pad = EP - E
    rows = jnp.concatenate([g_row, jnp.zeros((pad,), jnp.int32)])
    cols = jnp.concatenate([g_col, jnp.zeros((pad,), jnp.int32)])
    gv = jnp.concatenate([g_values, jnp.zeros((pad,), jnp.float32)])
    zeros = jnp.zeros((ROWS_PER_SUB, D), jnp.float32)

    atab, betab = _prep_tables(emb0, w1a, w1b)
    partials1, gv = _edge_kernels[0](
        _pack_words(atab), _pack_words(betab), rows, cols, gv, w2p, zeros)
    emb1, atab, betab = _mid_tables(partials1, w1a, w1b)
    partials2, gv = _edge_kernels[1](
        _pack_words(atab), _pack_words(betab), rows, cols, gv, w2p, zeros)
    return _final_mean(emb0, emb1, partials2)


# speedup vs baseline: 2.2682x; 1.0932x over previous
import functools

import jax
import jax.numpy as jnp
from jax import lax
from jax.experimental import pallas as pl
from jax.experimental.pallas import tpu as pltpu
from jax.experimental.pallas import tpu_sc as plsc

N_NODES = 10000
D = 128
E = 320000
N_LAYERS = 2

NW = 32
K = 64
NCH = 160
EP = NW * NCH * K
NGRP = K // 16
WPR = D // 2
CB = 2048
MCH = EP // CB
FW = 2

_ILV = plsc.PackFormat.INTERLEAVED


def _prep_body(x_ref, w1a_ref, w1b_ref, atab_ref, btab_ref):
    x = x_ref[...]
    a = jnp.dot(x, w1a_ref[...], preferred_element_type=jnp.float32)
    atab_ref[...] = a.astype(jnp.bfloat16)
    b = jnp.dot(x, w1b_ref[...], preferred_element_type=jnp.float32)
    btab_ref[...] = b.astype(jnp.bfloat16)


def _prep_tables(emb, w1a, w1b):
    bn = 2000
    return pl.pallas_call(
        _prep_body,
        grid=(N_NODES // bn,),
        in_specs=[pl.BlockSpec((bn, D), lambda i: (i, 0)),
                  pl.BlockSpec((D, D), lambda i: (0, 0)),
                  pl.BlockSpec((D, D), lambda i: (0, 0))],
        out_specs=[pl.BlockSpec((bn, D), lambda i: (i, 0)),
                   pl.BlockSpec((bn, D), lambda i: (i, 0))],
        out_shape=[jax.ShapeDtypeStruct((N_NODES, D), jnp.bfloat16),
                   jax.ShapeDtypeStruct((N_NODES, D), jnp.bfloat16)],
    )(emb, w1a, w1b)


def _final_body(e0_ref, e1_ref, e2_ref, out_ref):
    out_ref[...] = (e0_ref[...] + e1_ref[...] + e2_ref[...]) * (
        1.0 / (N_LAYERS + 1))


def _final_mean(emb0, emb1, emb2):
    bn = 2000
    return pl.pallas_call(
        _final_body,
        grid=(N_NODES // bn,),
        in_specs=[pl.BlockSpec((bn, D), lambda i: (i, 0)),
                  pl.BlockSpec((bn, D), lambda i: (i, 0)),
                  pl.BlockSpec((bn, D), lambda i: (i, 0))],
        out_specs=pl.BlockSpec((bn, D), lambda i: (i, 0)),
        out_shape=jax.ShapeDtypeStruct((N_NODES, D), jnp.float32),
    )(emb0, emb1, emb2)


def _pack_words(x_bf16):
    n, m = x_bf16.shape
    return jax.lax.bitcast_convert_type(
        x_bf16.reshape(n, m // 2, 2), jnp.int32)


def _make_z_kernel(scale):
    mesh = plsc.VectorSubcoreMesh(core_axis_name="c", subcore_axis_name="s")

    @functools.partial(
        pl.kernel, mesh=mesh,
        compiler_params=pltpu.CompilerParams(needs_layout_passes=False,
                                             use_tc_tiling_on_sc=False),
        out_type=jax.ShapeDtypeStruct((EP,), jnp.float32),
        scratch_types=[
            pltpu.VMEM((4, K), jnp.int32),
            pltpu.VMEM((4, K), jnp.int32),
            pltpu.VMEM((4, K), jnp.float32),
            pltpu.VMEM((2, K), jnp.float32),
            pltpu.VMEM((2, K, WPR), jnp.int32),
            pltpu.VMEM((2, K, WPR), jnp.int32),
            pltpu.VMEM((WPR,), jnp.int32),
            pltpu.SemaphoreType.DMA,
            pltpu.SemaphoreType.DMA,
            pltpu.SemaphoreType.DMA,
            pltpu.SemaphoreType.DMA,
            pltpu.SemaphoreType.DMA,
            pltpu.SemaphoreType.DMA,
        ],
    )
    def z_kernel(atab, btab, rows_hbm, cols_hbm, gv_hbm, w2_hbm,
                 gv_out, rows_v, cols_v, gv_v, gvn_v, S, T,
                 w2_v, sg0, sg1, si0, si1, sn0, sn1):
        cid = lax.axis_index("c")
        sid = lax.axis_index("s")
        w = cid * 16 + sid
        sem_g = (sg0, sg1)
        sem_i = (si0, si1)
        sem_n = (sn0, sn1)
        base0 = w * (NCH * K)

        pltpu.sync_copy(w2_hbm, w2_v)

        for b in range(2):
            pltpu.sync_copy(rows_hbm.at[pl.ds(base0 + b * K, K)], rows_v.at[b])
            pltpu.sync_copy(cols_hbm.at[pl.ds(base0 + b * K, K)], cols_v.at[b])
            pltpu.sync_copy(gv_hbm.at[pl.ds(base0 + b * K, K)], gv_v.at[b])
            pltpu.async_copy(atab.at[rows_v.at[b]], S.at[b], sem_g[b])
            pltpu.async_copy(btab.at[cols_v.at[b]], T.at[b], sem_g[b])

        erows = [lax.iota(jnp.int32, 16) + g * 16 for g in range(NGRP)]

        def quad_body(q, _carry):
            for j in range(4):
                p = j % 2
                ck = 4 * q + j
                base = base0 + ck * K
                Sb, Tb = S.at[p], T.at[p]
                pltpu.make_async_copy(atab.at[rows_v.at[j]], Sb,
                                      sem_g[p]).wait()
                pltpu.make_async_copy(btab.at[cols_v.at[j]], Tb,
                                      sem_g[p]).wait()
                @pl.when(ck >= 2)
                def _():
                    pltpu.make_async_copy(
                        gvn_v.at[p], gv_out.at[pl.ds(base, K)],
                        sem_n[p]).wait()
                @pl.when((q < NCH // 4 - 1) if j >= 2 else (q >= 0))
                def _():
                    nb = base + 2 * K
                    sl = (j + 2) % 4
                    pltpu.async_copy(rows_hbm.at[pl.ds(nb, K)],
                                     rows_v.at[sl], sem_i[p])
                    pltpu.async_copy(cols_hbm.at[pl.ds(nb, K)],
                                     cols_v.at[sl], sem_i[p])
                    pltpu.async_copy(gv_hbm.at[pl.ds(nb, K)],
                                     gv_v.at[sl], sem_i[p])

                def zbody(jj, accs):
                    out = list(accs)
                    for dd in range(2):
                        wd = jj * 2 + dd
                        wsplat = jnp.broadcast_to(wd, (16,))
                        w2p = plsc.bitcast(plsc.load_gather(w2_v, [wsplat]),
                                           jnp.bfloat16)
                        w2e, w2o = plsc.unpack(w2p, format=_ILV)
                        for g in range(NGRP):
                            ap = plsc.bitcast(
                                plsc.load_gather(Sb, [erows[g], wsplat]),
                                jnp.bfloat16)
                            bp = plsc.bitcast(
                                plsc.load_gather(Tb, [erows[g], wsplat]),
                                jnp.bfloat16)
                            ae, ao = plsc.unpack(ap, format=_ILV)
                            be, bo = plsc.unpack(bp, format=_ILV)
                            out[g] = (out[g]
                                      + jnp.maximum(ae + be, 0.0) * w2e
                                      + jnp.maximum(ao + bo, 0.0) * w2o)
                    return tuple(out)

                z0 = tuple(jnp.zeros((16,), jnp.float32) for _ in range(NGRP))
                zs = lax.fori_loop(0, WPR // 2, zbody, z0)

                gvb = gv_v.at[j]
                gvnb = gvn_v.at[p]
                for g in range(NGRP):
                    sig = 1.0 / (1.0 + jnp.exp(-zs[g]))
                    gvn = gvb[pl.ds(g * 16, 16)] * jnp.exp(sig * (-scale))
                    gvnb[pl.ds(g * 16, 16)] = gvn

                pltpu.async_copy(gvnb, gv_out.at[pl.ds(base, K)], sem_n[p])

                @pl.when((q < NCH // 4 - 1) if j >= 2 else (q >= 0))
                def _():
                    sl = (j + 2) % 4
                    pltpu.make_async_copy(rows_hbm.at[pl.ds(base, K)],
                                          rows_v.at[sl], sem_i[p]).wait()
                    pltpu.make_async_copy(cols_hbm.at[pl.ds(base, K)],
                                          cols_v.at[sl], sem_i[p]).wait()
                    pltpu.make_async_copy(gv_hbm.at[pl.ds(base, K)],
                                          gv_v.at[sl], sem_i[p]).wait()
                    pltpu.async_copy(atab.at[rows_v.at[sl]], Sb, sem_g[p])
                    pltpu.async_copy(btab.at[cols_v.at[sl]], Tb, sem_g[p])
            return 0

        lax.fori_loop(0, NCH // 4, quad_body, 0)

        for p in range(2):
            ck = NCH - 2 + p
            pltpu.make_async_copy(gvn_v.at[p],
                                  gv_out.at[pl.ds(base0 + ck * K, K)],
                                  sem_n[p]).wait()

    return z_kernel


_z_kernels = [_make_z_kernel(float(layer + 1)) for layer in range(N_LAYERS)]


def _make_msg_kernel():
    mesh = plsc.VectorSubcoreMesh(core_axis_name="c", subcore_axis_name="s")

    @functools.partial(
        pl.kernel, mesh=mesh,
        compiler_params=pltpu.CompilerParams(needs_layout_passes=False,
                                             use_tc_tiling_on_sc=False),
        out_type=jax.ShapeDtypeStruct((NW, N_NODES, 2 * FW), jnp.float32),
        scratch_types=[
            pltpu.VMEM((FW, N_NODES), jnp.int32),
            pltpu.VMEM((N_NODES, 2 * FW), jnp.float32),
            pltpu.VMEM((2, CB), jnp.int32),
            pltpu.VMEM((2, CB), jnp.int32),
            pltpu.VMEM((2, CB), jnp.float32),
            pltpu.SemaphoreType.DMA,
            pltpu.SemaphoreType.DMA,
        ],
    )
    def msg_kernel(embt, rows_hbm, cols_hbm, gv_hbm, zeros_hbm,
                   outp, embv, accv, rows_v, cols_v, gv_v, se0, se1):
        cid = lax.axis_index("c")
        sid = lax.axis_index("s")
        w = cid * 16 + sid
        sem = (se0, se1)

        pltpu.sync_copy(embt.at[pl.ds(FW * w, FW)], embv)
        pltpu.sync_copy(zeros_hbm, accv)

        pltpu.sync_copy(rows_hbm.at[pl.ds(0, CB)], rows_v.at[0])
        pltpu.sync_copy(cols_hbm.at[pl.ds(0, CB)], cols_v.at[0])
        pltpu.sync_copy(gv_hbm.at[pl.ds(0, CB)], gv_v.at[0])

        lanes = lax.iota(jnp.int32, 16)
        w0 = jnp.zeros((16,), jnp.int32)
        w1 = jnp.ones((16,), jnp.int32)
        k0 = jnp.full((16,), 0, jnp.int32)
        k1 = jnp.full((16,), 1, jnp.int32)
        k2 = jnp.full((16,), 2, jnp.int32)
        k3 = jnp.full((16,), 3, jnp.int32)

        def pair_body(q, _carry):
            for j in range(2):
                i = 2 * q + j
                p = j
                np_ = 1 - j

                @pl.when(i < MCH - 1)
                def _():
                    nb = (i + 1) * CB
                    pltpu.async_copy(rows_hbm.at[pl.ds(nb, CB)],
                                     rows_v.at[np_], sem[np_])
                    pltpu.async_copy(cols_hbm.at[pl.ds(nb, CB)],
                                     cols_v.at[np_], sem[np_])
                    pltpu.async_copy(gv_hbm.at[pl.ds(nb, CB)],
                                     gv_v.at[np_], sem[np_])

                rb, cb, gb = rows_v.at[p], cols_v.at[p], gv_v.at[p]

                @plsc.parallel_loop(0, CB // 16, unroll=2)
                def _(g):
                    eidx = lanes + g * 16
                    c16 = plsc.load_gather(cb, [eidx])
                    r16 = plsc.load_gather(rb, [eidx])
                    g16 = plsc.load_gather(gb, [eidx])
                    e0, e1 = plsc.unpack(
                        plsc.bitcast(plsc.load_gather(embv, [w0, c16]),
                                     jnp.bfloat16), format=_ILV)
                    e2, e3 = plsc.unpack(
                        plsc.bitcast(plsc.load_gather(embv, [w1, c16]),
                                     jnp.bfloat16), format=_ILV)
                    plsc.addupdate_scatter(accv, [r16, k0], e0 * g16)
                    plsc.addupdate_scatter(accv, [r16, k1], e1 * g16)
                    plsc.addupdate_scatter(accv, [r16, k2], e2 * g16)
                    plsc.addupdate_scatter(accv, [r16, k3], e3 * g16)

                @pl.when(i < MCH - 1)
                def _():
                    nb = (i + 1) * CB
                    pltpu.make_async_copy(rows_hbm.at[pl.ds(nb, CB)],
                                          rows_v.at[np_], sem[np_]).wait()
                    pltpu.make_async_copy(cols_hbm.at[pl.ds(nb, CB)],
                                          cols_v.at[np_], sem[np_]).wait()
                    pltpu.make_async_copy(gv_hbm.at[pl.ds(nb, CB)],
                                          gv_v.at[np_], sem[np_]).wait()
            return 0

        lax.fori_loop(0, MCH // 2, pair_body, 0)

        pltpu.sync_copy(accv, outp.at[w])

    return msg_kernel


_msg_kernel = _make_msg_kernel()


def kernel(user_emb, item_emb, g_values, W1, W2, g_row, g_col):
    emb0 = jnp.concatenate([user_emb, item_emb], axis=0)
    w1a, w1b = W1[:D], W1[D:]
    w2p = _pack_words(jnp.reshape(W2, (1, D)).astype(jnp.bfloat16))[0]
    pad = EP - E
    rows = jnp.concatenate([g_row, jnp.zeros((pad,), jnp.int32)])
    cols = jnp.concatenate([g_col, jnp.zeros((pad,), jnp.int32)])
    gv = jnp.concatenate([g_values, jnp.zeros((pad,), jnp.float32)])
    zeros4 = jnp.zeros((N_NODES, 2 * FW), jnp.float32)

    embs = [emb0]
    for layer in range(N_LAYERS):
        atab, btab = _prep_tables(embs[layer], w1a, w1b)
        gv = _z_kernels[layer](
            _pack_words(atab), _pack_words(btab), rows, cols, gv, w2p)
        embt = jnp.transpose(
            _pack_words(embs[layer].astype(jnp.bfloat16)))
        outp = _msg_kernel(embt, rows, cols, gv, zeros4)
        embs.append(
            jnp.transpose(outp, (1, 0, 2)).reshape(N_NODES, D))
    return _final_mean(embs[0], embs[1], embs[2])
